# baseline (speedup 1.0000x reference)
import functools

import jax
import jax.numpy as jnp
from jax import lax
from jax.experimental import pallas as pl
from jax.experimental.pallas import tpu as pltpu
from jax.experimental.pallas import tpu_sc as plsc

B, L, DC, DE = 16, 4096, 128, 128
NC, NS = 2, 16
NW = NC * NS
HALF = L // 2
BATCH = 64
NBATCH = HALF // BATCH
CTX_BLK = 4096
NBUF = 8


def _sc_body(ex_hbm, mask_hbm, oex_hbm,
             mask_v, valid_v, idx_v, rows_v, sem_m, *sems):
    c = lax.axis_index("c")
    s = lax.axis_index("s")
    wid = s * NC + c
    b = wid // 2
    h = wid % 2
    out_base = b * L + h * HALF

    iota = lax.iota(jnp.int32, 16)

    pltpu.async_copy(mask_hbm.at[b], mask_v, sem_m).wait()

    def comp_body(i, off):
        base = i * 64
        sorted_pos, counts = [], []
        for u in range(4):
            moff = pl.multiple_of(base + u * 16, 16)
            m = mask_v[pl.ds(moff, 16)]
            keys = iota + (1 - m) * 16
            _, ps = plsc.sort_key_val(keys, iota + (base + u * 16))
            sorted_pos.append(ps)
            counts.append(plsc.all_reduce_population_count(m > 0))
        for u in range(4):
            valid_v[pl.ds(off, 16)] = sorted_pos[u]
            off = off + counts[u][0]
        return off

    nv = lax.fori_loop(0, L // 64, comp_body, jnp.int32(0))

    @pl.when(nv == 0)
    def _():
        valid_v[pl.ds(0, 16)] = iota * 0

    nvb = jnp.broadcast_to(jnp.maximum(nv, 1), (16,))

    def build_idx(g, slot):
        lbase = h * HALF + g * BATCH
        for j in range(BATCH // 16):
            lvec = iota + (lbase + j * 16)
            colv = plsc.load_gather(valid_v, [lax.rem(lvec, nvb)])
            idx_v[slot, pl.ds(j * 16, 16)] = colv + b * L

    sem_g = sems[:NBUF]
    sem_w = sems[NBUF:]

    def wait_gather(slot):
        pltpu.make_async_copy(ex_hbm.at[idx_v.at[slot]], rows_v.at[slot],
                              sem_g[slot]).wait()

    def wait_write(slot):
        pltpu.make_async_copy(rows_v.at[slot], oex_hbm.at[pl.ds(0, BATCH)],
                              sem_w[slot]).wait()

    def gath_block(blk, carry):
        g0 = blk * NBUF

        @pl.when(blk > 0)
        def _():
            for u in range(NBUF):
                wait_write(u)

        def eff(g):
            return jnp.where(h == 0, g, NBATCH - 1 - g)

        for u in range(NBUF):
            build_idx(eff(g0 + u), u)
            pltpu.async_copy(ex_hbm.at[idx_v.at[u]], rows_v.at[u], sem_g[u])
        for u in range(NBUF):
            wait_gather(u)
            pltpu.async_copy(
                rows_v.at[u],
                oex_hbm.at[pl.ds(out_base + eff(g0 + u) * BATCH, BATCH)],
                sem_w[u])
        return carry

    lax.fori_loop(0, NBATCH // NBUF, gath_block, 0)
    for u in range(NBUF):
        wait_write(u)


_flatten_sc = functools.partial(
    pl.kernel,
    out_type=jax.ShapeDtypeStruct((B * L, DE), jnp.float32),
    mesh=plsc.VectorSubcoreMesh(core_axis_name="c", subcore_axis_name="s"),
    compiler_params=pltpu.CompilerParams(needs_layout_passes=False),
    scratch_types=[
        pltpu.VMEM((L,), jnp.int32),
        pltpu.VMEM((L + 16,), jnp.int32),
        pltpu.VMEM((NBUF, BATCH), jnp.int32),
        pltpu.VMEM((NBUF, BATCH, DE), jnp.float32),
        pltpu.SemaphoreType.DMA,
    ] + [pltpu.SemaphoreType.DMA] * (2 * NBUF),
)(_sc_body)


CTX_ROWS = 4


def _ctx_body(ctx_ref, out_ref):
    i = pl.program_id(0)
    for k in range(CTX_ROWS):
        row = ctx_ref[pl.ds(i * CTX_ROWS + k, 1), :]
        out_ref[pl.ds(k * L, L), :] = jnp.broadcast_to(row, (L, DC))


_flatten_ctx_tc = pl.pallas_call(
    _ctx_body,
    grid=(B // CTX_ROWS,),
    in_specs=[pl.BlockSpec((B, DC), lambda i: (0, 0))],
    out_specs=pl.BlockSpec((CTX_ROWS * L, DC), lambda i: (i, 0)),
    out_shape=jax.ShapeDtypeStruct((B * L, DC), jnp.float32),
)


def kernel(context_features, example_features, mask):
    ex_flat = example_features.reshape(B * L, DE)
    mask_i = mask.astype(jnp.int32)
    flat_ex = _flatten_sc(ex_flat, mask_i)
    flat_ctx = _flatten_ctx_tc(context_features)
    return flat_ctx, flat_ex


# speedup vs baseline: 1.0089x; 1.0089x over previous
import functools

import jax
import jax.numpy as jnp
from jax import lax
from jax.experimental import pallas as pl
from jax.experimental.pallas import tpu as pltpu
from jax.experimental.pallas import tpu_sc as plsc

B, L, DC, DE = 16, 4096, 128, 128
NC, NS = 2, 16
NW = NC * NS
HALF = L // 2
BATCH = 64
NBATCH = HALF // BATCH
CTX_BLK = 4096
NBUF = 8


def _sc_body(ex_hbm, mask_hbm, oex_hbm,
             mask_v, valid_v, idx_v, rows_v, sem_m, *sems):
    c = lax.axis_index("c")
    s = lax.axis_index("s")
    wid = s * NC + c
    b = wid // 2
    h = wid % 2
    out_base = b * L + h * HALF

    iota = lax.iota(jnp.int32, 16)

    pltpu.async_copy(mask_hbm.at[b], mask_v, sem_m).wait()

    def comp_body(i, off):
        base = i * 64
        sorted_pos, counts = [], []
        for u in range(4):
            moff = pl.multiple_of(base + u * 16, 16)
            m = mask_v[pl.ds(moff, 16)]
            keys = iota + (1 - m) * 16
            _, ps = plsc.sort_key_val(keys, iota + (base + u * 16))
            sorted_pos.append(ps)
            counts.append(plsc.all_reduce_population_count(m > 0))
        for u in range(4):
            valid_v[pl.ds(off, 16)] = sorted_pos[u]
            off = off + counts[u][0]
        return off

    nv = lax.fori_loop(0, L // 64, comp_body, jnp.int32(0))

    @pl.when(nv == 0)
    def _():
        valid_v[pl.ds(0, 16)] = iota * 0

    nvb = jnp.broadcast_to(jnp.maximum(nv, 1), (16,))

    def build_idx(g, slot):
        lbase = h * HALF + g * BATCH
        for j in range(BATCH // 16):
            lvec = iota + (lbase + j * 16)
            colv = plsc.load_gather(valid_v, [lax.rem(lvec, nvb)])
            idx_v[slot, pl.ds(j * 16, 16)] = colv + b * L

    sem_g = sems[:NBUF]
    sem_w = sems[NBUF:]

    def wait_gather(slot):
        pltpu.make_async_copy(ex_hbm.at[idx_v.at[slot]], rows_v.at[slot],
                              sem_g[slot]).wait()

    def wait_write(slot):
        pltpu.make_async_copy(rows_v.at[slot], oex_hbm.at[pl.ds(0, BATCH)],
                              sem_w[slot]).wait()

    def gath_block(blk, carry):
        g0 = blk * NBUF

        @pl.when(blk > 0)
        def _():
            for u in range(NBUF):
                wait_write(u)

        def eff(g):
            return jnp.where(h == 0, g, NBATCH - 1 - g)

        for u in range(NBUF):
            build_idx(eff(g0 + u), u)
            pltpu.async_copy(ex_hbm.at[idx_v.at[u]], rows_v.at[u], sem_g[u])
        for u in range(NBUF):
            wait_gather(u)
            pltpu.async_copy(
                rows_v.at[u],
                oex_hbm.at[pl.ds(out_base + eff(g0 + u) * BATCH, BATCH)],
                sem_w[u])
        return carry

    lax.fori_loop(0, NBATCH // NBUF, gath_block, 0)
    for u in range(NBUF):
        wait_write(u)


_flatten_sc = functools.partial(
    pl.kernel,
    out_type=jax.ShapeDtypeStruct((B * L, DE), jnp.float32),
    mesh=plsc.VectorSubcoreMesh(core_axis_name="c", subcore_axis_name="s"),
    compiler_params=pltpu.CompilerParams(needs_layout_passes=False),
    scratch_types=[
        pltpu.VMEM((L,), jnp.int32),
        pltpu.VMEM((L + 16,), jnp.int32),
        pltpu.VMEM((NBUF, BATCH), jnp.int32),
        pltpu.VMEM((NBUF, BATCH, DE), jnp.float32),
        pltpu.SemaphoreType.DMA,
    ] + [pltpu.SemaphoreType.DMA] * (2 * NBUF),
)(_sc_body)


CTX_ROWS = 2


def _ctx_body(ctx_ref, out_ref):
    i = pl.program_id(0)
    for k in range(CTX_ROWS):
        row = ctx_ref[pl.ds(i * CTX_ROWS + k, 1), :]
        out_ref[pl.ds(k * L, L), :] = jnp.broadcast_to(row, (L, DC))


_flatten_ctx_tc = pl.pallas_call(
    _ctx_body,
    grid=(B // CTX_ROWS,),
    in_specs=[pl.BlockSpec((B, DC), lambda i: (0, 0))],
    out_specs=pl.BlockSpec((CTX_ROWS * L, DC), lambda i: (i, 0)),
    out_shape=jax.ShapeDtypeStruct((B * L, DC), jnp.float32),
)


def kernel(context_features, example_features, mask):
    ex_flat = example_features.reshape(B * L, DE)
    mask_i = mask.astype(jnp.int32)
    flat_ex = _flatten_sc(ex_flat, mask_i)
    flat_ctx = _flatten_ctx_tc(context_features)
    return flat_ctx, flat_ex


# speedup vs baseline: 1.0100x; 1.0010x over previous
import functools

import jax
import jax.numpy as jnp
from jax import lax
from jax.experimental import pallas as pl
from jax.experimental.pallas import tpu as pltpu
from jax.experimental.pallas import tpu_sc as plsc

B, L, DC, DE = 16, 4096, 128, 128
NC, NS = 2, 16
NW = NC * NS
HALF = L // 2
BATCH = 64
NBATCH = HALF // BATCH
CTX_BLK = 4096
NBUF = 8


def _sc_body(ex_hbm, mask_hbm, oex_hbm,
             mask_v, valid_v, idx_v, rows_v, sem_m, *sems):
    c = lax.axis_index("c")
    s = lax.axis_index("s")
    wid = s * NC + c
    b = wid // 2
    h = wid % 2
    out_base = b * L + h * HALF

    iota = lax.iota(jnp.int32, 16)

    pltpu.async_copy(mask_hbm.at[b], mask_v, sem_m).wait()

    def comp_body(i, off):
        base = i * 64
        sorted_pos, counts = [], []
        for u in range(4):
            moff = pl.multiple_of(base + u * 16, 16)
            m = mask_v[pl.ds(moff, 16)]
            keys = iota + (1 - m) * 16
            _, ps = plsc.sort_key_val(keys, iota + (base + u * 16))
            sorted_pos.append(ps)
            counts.append(plsc.all_reduce_population_count(m > 0))
        for u in range(4):
            valid_v[pl.ds(off, 16)] = sorted_pos[u]
            off = off + counts[u][0]
        return off

    nv = lax.fori_loop(0, L // 64, comp_body, jnp.int32(0))

    @pl.when(nv == 0)
    def _():
        valid_v[pl.ds(0, 16)] = iota * 0

    nvb = jnp.broadcast_to(jnp.maximum(nv, 1), (16,))

    def build_idx(g, slot):
        lbase = h * HALF + g * BATCH
        for j in range(BATCH // 16):
            lvec = iota + (lbase + j * 16)
            colv = plsc.load_gather(valid_v, [lax.rem(lvec, nvb)])
            idx_v[slot, pl.ds(j * 16, 16)] = colv + b * L

    sem_g = sems[:NBUF]
    sem_w = sems[NBUF:]

    def wait_gather(slot):
        pltpu.make_async_copy(ex_hbm.at[idx_v.at[slot]], rows_v.at[slot],
                              sem_g[slot]).wait()

    def wait_write(slot):
        pltpu.make_async_copy(rows_v.at[slot], oex_hbm.at[pl.ds(0, BATCH)],
                              sem_w[slot]).wait()

    def gath_block(blk, carry):
        g0 = blk * NBUF

        @pl.when(blk > 0)
        def _():
            for u in range(NBUF):
                wait_write(u)

        def eff(g):
            return lax.rem(g + h * (NBATCH // 2), NBATCH)

        for u in range(NBUF):
            build_idx(eff(g0 + u), u)
            pltpu.async_copy(ex_hbm.at[idx_v.at[u]], rows_v.at[u], sem_g[u])
        for u in range(NBUF):
            wait_gather(u)
            pltpu.async_copy(
                rows_v.at[u],
                oex_hbm.at[pl.ds(out_base + eff(g0 + u) * BATCH, BATCH)],
                sem_w[u])
        return carry

    lax.fori_loop(0, NBATCH // NBUF, gath_block, 0)
    for u in range(NBUF):
        wait_write(u)


_flatten_sc = functools.partial(
    pl.kernel,
    out_type=jax.ShapeDtypeStruct((B * L, DE), jnp.float32),
    mesh=plsc.VectorSubcoreMesh(core_axis_name="c", subcore_axis_name="s"),
    compiler_params=pltpu.CompilerParams(needs_layout_passes=False),
    scratch_types=[
        pltpu.VMEM((L,), jnp.int32),
        pltpu.VMEM((L + 16,), jnp.int32),
        pltpu.VMEM((NBUF, BATCH), jnp.int32),
        pltpu.VMEM((NBUF, BATCH, DE), jnp.float32),
        pltpu.SemaphoreType.DMA,
    ] + [pltpu.SemaphoreType.DMA] * (2 * NBUF),
)(_sc_body)


CTX_ROWS = 2


def _ctx_body(ctx_ref, out_ref):
    i = pl.program_id(0)
    for k in range(CTX_ROWS):
        row = ctx_ref[pl.ds(i * CTX_ROWS + k, 1), :]
        out_ref[pl.ds(k * L, L), :] = jnp.broadcast_to(row, (L, DC))


_flatten_ctx_tc = pl.pallas_call(
    _ctx_body,
    grid=(B // CTX_ROWS,),
    in_specs=[pl.BlockSpec((B, DC), lambda i: (0, 0))],
    out_specs=pl.BlockSpec((CTX_ROWS * L, DC), lambda i: (i, 0)),
    out_shape=jax.ShapeDtypeStruct((B * L, DC), jnp.float32),
)


def kernel(context_features, example_features, mask):
    ex_flat = example_features.reshape(B * L, DE)
    mask_i = mask.astype(jnp.int32)
    flat_ex = _flatten_sc(ex_flat, mask_i)
    flat_ctx = _flatten_ctx_tc(context_features)
    return flat_ctx, flat_ex


# speedup vs baseline: 1.0150x; 1.0049x over previous
import functools

import jax
import jax.numpy as jnp
from jax import lax
from jax.experimental import pallas as pl
from jax.experimental.pallas import tpu as pltpu
from jax.experimental.pallas import tpu_sc as plsc

B, L, DC, DE = 16, 4096, 128, 128
NC, NS = 2, 16
NW = NC * NS
HALF = L // 2
BATCH = 64
NBATCH = HALF // BATCH
CTX_BLK = 4096
NBUF = 8


def _sc_body(ex_hbm, mask_hbm, oex_hbm,
             mask_v, valid_v, idx_v, rows_v, sem_m, *sems):
    c = lax.axis_index("c")
    s = lax.axis_index("s")
    wid = c * NS + s
    b = wid // 2
    h = wid % 2
    out_base = b * L + h * HALF

    iota = lax.iota(jnp.int32, 16)

    pltpu.async_copy(mask_hbm.at[b], mask_v, sem_m).wait()

    def comp_body(i, off):
        base = i * 64
        sorted_pos, counts = [], []
        for u in range(4):
            moff = pl.multiple_of(base + u * 16, 16)
            m = mask_v[pl.ds(moff, 16)]
            keys = iota + (1 - m) * 16
            _, ps = plsc.sort_key_val(keys, iota + (base + u * 16))
            sorted_pos.append(ps)
            counts.append(plsc.all_reduce_population_count(m > 0))
        for u in range(4):
            valid_v[pl.ds(off, 16)] = sorted_pos[u]
            off = off + counts[u][0]
        return off

    nv = lax.fori_loop(0, L // 64, comp_body, jnp.int32(0))

    @pl.when(nv == 0)
    def _():
        valid_v[pl.ds(0, 16)] = iota * 0

    nvb = jnp.broadcast_to(jnp.maximum(nv, 1), (16,))

    def build_idx(g, slot):
        lbase = h * HALF + g * BATCH
        for j in range(BATCH // 16):
            lvec = iota + (lbase + j * 16)
            colv = plsc.load_gather(valid_v, [lax.rem(lvec, nvb)])
            idx_v[slot, pl.ds(j * 16, 16)] = colv + b * L

    sem_g = sems[:NBUF]
    sem_w = sems[NBUF:]

    def wait_gather(slot):
        pltpu.make_async_copy(ex_hbm.at[idx_v.at[slot]], rows_v.at[slot],
                              sem_g[slot]).wait()

    def wait_write(slot):
        pltpu.make_async_copy(rows_v.at[slot], oex_hbm.at[pl.ds(0, BATCH)],
                              sem_w[slot]).wait()

    def gath_block(blk, carry):
        g0 = blk * NBUF

        @pl.when(blk > 0)
        def _():
            for u in range(NBUF):
                wait_write(u)

        def eff(g):
            return lax.rem(g + h * (NBATCH // 2), NBATCH)

        for u in range(NBUF):
            build_idx(eff(g0 + u), u)
            pltpu.async_copy(ex_hbm.at[idx_v.at[u]], rows_v.at[u], sem_g[u])
        for u in range(NBUF):
            wait_gather(u)
            pltpu.async_copy(
                rows_v.at[u],
                oex_hbm.at[pl.ds(out_base + eff(g0 + u) * BATCH, BATCH)],
                sem_w[u])
        return carry

    lax.fori_loop(0, NBATCH // NBUF, gath_block, 0)
    for u in range(NBUF):
        wait_write(u)


_flatten_sc = functools.partial(
    pl.kernel,
    out_type=jax.ShapeDtypeStruct((B * L, DE), jnp.float32),
    mesh=plsc.VectorSubcoreMesh(core_axis_name="c", subcore_axis_name="s"),
    compiler_params=pltpu.CompilerParams(needs_layout_passes=False),
    scratch_types=[
        pltpu.VMEM((L,), jnp.int32),
        pltpu.VMEM((L + 16,), jnp.int32),
        pltpu.VMEM((NBUF, BATCH), jnp.int32),
        pltpu.VMEM((NBUF, BATCH, DE), jnp.float32),
        pltpu.SemaphoreType.DMA,
    ] + [pltpu.SemaphoreType.DMA] * (2 * NBUF),
)(_sc_body)


CTX_ROWS = 2


def _ctx_body(ctx_ref, out_ref):
    i = pl.program_id(0)
    for k in range(CTX_ROWS):
        row = ctx_ref[pl.ds(i * CTX_ROWS + k, 1), :]
        out_ref[pl.ds(k * L, L), :] = jnp.broadcast_to(row, (L, DC))


_flatten_ctx_tc = pl.pallas_call(
    _ctx_body,
    grid=(B // CTX_ROWS,),
    in_specs=[pl.BlockSpec((B, DC), lambda i: (0, 0))],
    out_specs=pl.BlockSpec((CTX_ROWS * L, DC), lambda i: (i, 0)),
    out_shape=jax.ShapeDtypeStruct((B * L, DC), jnp.float32),
)


def kernel(context_features, example_features, mask):
    ex_flat = example_features.reshape(B * L, DE)
    mask_i = mask.astype(jnp.int32)
    flat_ex = _flatten_sc(ex_flat, mask_i)
    flat_ctx = _flatten_ctx_tc(context_features)
    return flat_ctx, flat_ex
